# Initial kernel scaffold; baseline (speedup 1.0000x reference)
import jax
import jax.numpy as jnp
from jax.experimental import pallas as pl


def kernel(edge_attr, W0, W1, W2):
    raise NotImplementedError("write your pallas kernel here")


# speedup vs baseline: 1.1759x; 1.1759x over previous
import functools

import jax
import jax.numpy as jnp
from jax import lax
from jax.experimental import pallas as pl
from jax.experimental.pallas import tpu as pltpu
from jax.experimental.pallas import tpu_sc as plsc

N = 800000
D = 64
CHUNK = 640
NUM_CHUNKS = N // CHUNK
NW = 32
MAX_J = (NUM_CHUNKS + NW - 1) // NW
GROUPS = CHUNK // 16


def _body(ea_hbm, w0_hbm, w1_hbm, w2_hbm, out_hbm,
          w0_v, w1_v, w2_v, lut_v, edges_v, rows_v):
    wid = lax.axis_index("s") * 2 + lax.axis_index("c")

    pltpu.sync_copy(w0_hbm, w0_v)
    pltpu.sync_copy(w1_hbm, w1_v)
    pltpu.sync_copy(w2_hbm, w2_v)

    for r in range(60):
        r0, r1, r2 = r // 12, (r // 2) % 6, r % 2
        for cg in range(4):
            lut_v[pl.ds(r * 64 + cg * 16, 16)] = (
                w0_v[pl.ds(r0 * 64 + cg * 16, 16)]
                + w1_v[pl.ds(r1 * 64 + cg * 16, 16)]
                + w2_v[pl.ds(r2 * 64 + cg * 16, 16)])

    iota = lax.iota(jnp.int32, 16)
    idx3 = iota * 3
    iota64 = iota * 64

    def chunk_body(j, carry):
        cid = wid + j * NW

        @pl.when(cid < NUM_CHUNKS)
        def _():
            pltpu.sync_copy(ea_hbm.at[pl.ds(cid * (CHUNK * 3), CHUNK * 3)],
                            edges_v)

            def group_body(g, c2):
                base = g * 48
                e0 = plsc.load_gather(edges_v, [idx3 + base])
                e1 = plsc.load_gather(edges_v, [idx3 + (base + 1)])
                e2 = plsc.load_gather(edges_v, [idx3 + (base + 2)])
                e0 = jnp.clip(e0, 0, 4)
                e1 = jnp.clip(e1, 0, 5)
                e2 = jnp.clip(e2, 0, 1)
                src = (e0 * 12 + e1 * 2 + e2) * 64
                dstb = iota64 + g * (16 * 64)
                for c in range(64):
                    vals = plsc.load_gather(lut_v, [src + c])
                    plsc.store_scatter(rows_v, [dstb + c], vals)
                return c2

            lax.fori_loop(0, GROUPS, group_body, 0)
            pltpu.sync_copy(rows_v,
                            out_hbm.at[pl.ds(cid * (CHUNK * D), CHUNK * D)])

        return carry

    lax.fori_loop(0, MAX_J, chunk_body, 0)


_sc_lookup = functools.partial(
    pl.kernel,
    mesh=plsc.VectorSubcoreMesh(core_axis_name="c", subcore_axis_name="s"),
    out_type=jax.ShapeDtypeStruct((N * D,), jnp.float32),
    compiler_params=pltpu.CompilerParams(needs_layout_passes=False),
    scratch_types=[
        pltpu.VMEM((5 * 64,), jnp.float32),
        pltpu.VMEM((6 * 64,), jnp.float32),
        pltpu.VMEM((2 * 64,), jnp.float32),
        pltpu.VMEM((60 * 64,), jnp.float32),
        pltpu.VMEM((CHUNK * 3,), jnp.int32),
        pltpu.VMEM((CHUNK * D,), jnp.float32),
    ],
)(_body)


def kernel(edge_attr, W0, W1, W2):
    ea = edge_attr.astype(jnp.int32).reshape(-1)
    out = _sc_lookup(ea, W0.reshape(-1), W1.reshape(-1), W2.reshape(-1))
    return out.reshape(N, D)


# speedup vs baseline: 1.8545x; 1.5771x over previous
import functools

import jax
import jax.numpy as jnp
from jax import lax
from jax.experimental import pallas as pl
from jax.experimental.pallas import tpu as pltpu
from jax.experimental.pallas import tpu_sc as plsc

N = 800000
D = 64
CHUNK = 640
NUM_CHUNKS = N // CHUNK
NW = 32
MAX_T = (NUM_CHUNKS + NW - 1) // NW
GROUPS = CHUNK // 16
IDX_ROWS = CHUNK // 128


def _body(ea_hbm, w0_hbm, w1_hbm, w2_hbm, out_hbm,
          w0_v, w1_v, w2_v, lut_v, lut_sh, edges_v, idx_v,
          rows0, rows1, semg, semo0, semo1):
    cidm = lax.axis_index("c")
    sid = lax.axis_index("s")
    wid = sid * 2 + cidm

    @pl.when(sid == 0)
    def _():
        pltpu.sync_copy(w0_hbm, w0_v)
        pltpu.sync_copy(w1_hbm, w1_v)
        pltpu.sync_copy(w2_hbm, w2_v)
        for r in range(60):
            r0, r1, r2 = r // 12, (r // 2) % 6, r % 2
            for cg in range(4):
                lut_v[pl.ds(r * 64 + cg * 16, 16)] = (
                    w0_v[pl.ds(r0 * 64 + cg * 16, 16)]
                    + w1_v[pl.ds(r1 * 64 + cg * 16, 16)]
                    + w2_v[pl.ds(r2 * 64 + cg * 16, 16)])
        for r in range(60):
            pltpu.sync_copy(lut_v.at[pl.ds(r * 64, 64)], lut_sh.at[r])

    plsc.subcore_barrier()

    iota = lax.iota(jnp.int32, 16)
    idx3 = iota * 3

    def do_chunk(t, rows_v, semo):
        cid = wid + t * NW

        @pl.when(cid < NUM_CHUNKS)
        def _():
            @pl.when(t >= 2)
            def _():
                pltpu.make_async_copy(
                    rows_v, out_hbm.at[pl.ds(0, CHUNK)], semo).wait()

            pltpu.sync_copy(ea_hbm.at[pl.ds(cid * (CHUNK * 3), CHUNK * 3)],
                            edges_v)

            def group_body(g, c2):
                base = g * 48
                e0 = plsc.load_gather(edges_v, [idx3 + base])
                e1 = plsc.load_gather(edges_v, [idx3 + (base + 1)])
                e2 = plsc.load_gather(edges_v, [idx3 + (base + 2)])
                e0 = jnp.clip(e0, 0, 4)
                e1 = jnp.clip(e1, 0, 5)
                e2 = jnp.clip(e2, 0, 1)
                src = e0 * 12 + e1 * 2 + e2
                flat = iota + g * 16
                plsc.store_scatter(idx_v, [flat >> 7, flat & 127], src)
                return c2

            lax.fori_loop(0, GROUPS, group_body, 0)

            gathers = [
                pltpu.async_copy(lut_sh.at[idx_v.at[b]],
                                 rows_v.at[pl.ds(b * 128, 128)], semg)
                for b in range(IDX_ROWS)]
            for g in gathers:
                g.wait()

            pltpu.async_copy(rows_v, out_hbm.at[pl.ds(cid * CHUNK, CHUNK)],
                             semo)

        return cid < NUM_CHUNKS

    def chunk_body(j, carry):
        do_chunk(2 * j, rows0, semo0)
        do_chunk(2 * j + 1, rows1, semo1)
        return carry

    lax.fori_loop(0, MAX_T // 2, chunk_body, 0)

    nt = (NUM_CHUNKS - wid + NW - 1) // NW

    @pl.when(nt >= 1)
    def _():
        pltpu.make_async_copy(rows0, out_hbm.at[pl.ds(0, CHUNK)], semo0).wait()

    @pl.when(nt >= 2)
    def _():
        pltpu.make_async_copy(rows1, out_hbm.at[pl.ds(0, CHUNK)], semo1).wait()


_sc_lookup = functools.partial(
    pl.kernel,
    mesh=plsc.VectorSubcoreMesh(core_axis_name="c", subcore_axis_name="s"),
    out_type=jax.ShapeDtypeStruct((N, D), jnp.float32),
    compiler_params=pltpu.CompilerParams(needs_layout_passes=False,
                                         use_tc_tiling_on_sc=False),
    scratch_types=[
        pltpu.VMEM((5 * 64,), jnp.float32),
        pltpu.VMEM((6 * 64,), jnp.float32),
        pltpu.VMEM((2 * 64,), jnp.float32),
        pltpu.VMEM((60 * 64,), jnp.float32),
        pltpu.VMEM_SHARED((60, 64), jnp.float32),
        pltpu.VMEM((CHUNK * 3,), jnp.int32),
        pltpu.VMEM((IDX_ROWS, 128), jnp.int32),
        pltpu.VMEM((CHUNK, D), jnp.float32),
        pltpu.VMEM((CHUNK, D), jnp.float32),
        pltpu.SemaphoreType.DMA,
        pltpu.SemaphoreType.DMA,
        pltpu.SemaphoreType.DMA,
    ],
)(_body)


def kernel(edge_attr, W0, W1, W2):
    ea = edge_attr.astype(jnp.int32).reshape(-1)
    return _sc_lookup(ea, W0.reshape(-1), W1.reshape(-1), W2.reshape(-1))


# speedup vs baseline: 8.3533x; 4.5044x over previous
import functools

import jax
import jax.numpy as jnp
from jax import lax
from jax.experimental import pallas as pl
from jax.experimental.pallas import tpu as pltpu
from jax.experimental.pallas import tpu_sc as plsc

N = 800000
D = 64
CHUNK = 640
NUM_CHUNKS = N // CHUNK
NW = 32
MAX_T = (NUM_CHUNKS + NW - 1) // NW
GROUPS = CHUNK // 16
IDX_ROWS = CHUNK // 128


def _body(e0_hbm, e1_hbm, e2_hbm, w0_hbm, w1_hbm, w2_hbm, out_hbm,
          w0_v, w1_v, w2_v, lut_v, lut_sh, edges_v, idx_v,
          rows0, rows1, semg, semo0, semo1):
    cidm = lax.axis_index("c")
    sid = lax.axis_index("s")
    wid = sid * 2 + cidm

    @pl.when(sid == 0)
    def _():
        pltpu.sync_copy(w0_hbm, w0_v)
        pltpu.sync_copy(w1_hbm, w1_v)
        pltpu.sync_copy(w2_hbm, w2_v)
        for r in range(60):
            r0, r1, r2 = r // 12, (r // 2) % 6, r % 2
            for cg in range(4):
                lut_v[pl.ds(r * 64 + cg * 16, 16)] = (
                    w0_v[pl.ds(r0 * 64 + cg * 16, 16)]
                    + w1_v[pl.ds(r1 * 64 + cg * 16, 16)]
                    + w2_v[pl.ds(r2 * 64 + cg * 16, 16)])
        for r in range(60):
            pltpu.sync_copy(lut_v.at[pl.ds(r * 64, 64)], lut_sh.at[r])

    plsc.subcore_barrier()

    iota = lax.iota(jnp.int32, 16)

    def do_chunk(t, rows_v, semo):
        cid = wid + t * NW

        @pl.when(cid < NUM_CHUNKS)
        def _():
            @pl.when(t >= 2)
            def _():
                pltpu.make_async_copy(
                    rows_v, out_hbm.at[pl.ds(0, CHUNK)], semo).wait()

            pltpu.sync_copy(e0_hbm.at[pl.ds(cid * CHUNK, CHUNK)],
                            edges_v.at[pl.ds(0, CHUNK)])
            pltpu.sync_copy(e1_hbm.at[pl.ds(cid * CHUNK, CHUNK)],
                            edges_v.at[pl.ds(CHUNK, CHUNK)])
            pltpu.sync_copy(e2_hbm.at[pl.ds(cid * CHUNK, CHUNK)],
                            edges_v.at[pl.ds(2 * CHUNK, CHUNK)])

            def group_body(g, c2):
                base = g * 16
                e0 = edges_v[pl.ds(base, 16)]
                e1 = edges_v[pl.ds(CHUNK + base, 16)]
                e2 = edges_v[pl.ds(2 * CHUNK + base, 16)]
                e0 = jnp.clip(e0, 0, 4)
                e1 = jnp.clip(e1, 0, 5)
                e2 = jnp.clip(e2, 0, 1)
                src = e0 * 12 + e1 * 2 + e2
                flat = iota + g * 16
                plsc.store_scatter(idx_v, [flat >> 7, flat & 127], src)
                return c2

            lax.fori_loop(0, GROUPS, group_body, 0)

            gathers = [
                pltpu.async_copy(lut_sh.at[idx_v.at[b]],
                                 rows_v.at[pl.ds(b * 128, 128)], semg)
                for b in range(IDX_ROWS)]
            for g in gathers:
                g.wait()

            pltpu.async_copy(rows_v, out_hbm.at[pl.ds(cid * CHUNK, CHUNK)],
                             semo)

        return cid < NUM_CHUNKS

    def chunk_body(j, carry):
        do_chunk(2 * j, rows0, semo0)
        do_chunk(2 * j + 1, rows1, semo1)
        return carry

    lax.fori_loop(0, MAX_T // 2, chunk_body, 0)

    nt = (NUM_CHUNKS - wid + NW - 1) // NW

    @pl.when(nt >= 1)
    def _():
        pltpu.make_async_copy(rows0, out_hbm.at[pl.ds(0, CHUNK)], semo0).wait()

    @pl.when(nt >= 2)
    def _():
        pltpu.make_async_copy(rows1, out_hbm.at[pl.ds(0, CHUNK)], semo1).wait()


_sc_lookup = functools.partial(
    pl.kernel,
    mesh=plsc.VectorSubcoreMesh(core_axis_name="c", subcore_axis_name="s"),
    out_type=jax.ShapeDtypeStruct((N, D), jnp.float32),
    compiler_params=pltpu.CompilerParams(needs_layout_passes=False,
                                         use_tc_tiling_on_sc=False),
    scratch_types=[
        pltpu.VMEM((5 * 64,), jnp.float32),
        pltpu.VMEM((6 * 64,), jnp.float32),
        pltpu.VMEM((2 * 64,), jnp.float32),
        pltpu.VMEM((60 * 64,), jnp.float32),
        pltpu.VMEM_SHARED((60, 64), jnp.float32),
        pltpu.VMEM((CHUNK * 3,), jnp.int32),
        pltpu.VMEM((IDX_ROWS, 128), jnp.int32),
        pltpu.VMEM((CHUNK, D), jnp.float32),
        pltpu.VMEM((CHUNK, D), jnp.float32),
        pltpu.SemaphoreType.DMA,
        pltpu.SemaphoreType.DMA,
        pltpu.SemaphoreType.DMA,
    ],
)(_body)


def kernel(edge_attr, W0, W1, W2):
    ea = edge_attr.astype(jnp.int32)
    return _sc_lookup(ea[:, 0], ea[:, 1], ea[:, 2],
                      W0.reshape(-1), W1.reshape(-1), W2.reshape(-1))
